# baseline (speedup 1.0000x reference)
import functools

import jax
import jax.numpy as jnp
from jax import lax
from jax.experimental import pallas as pl
from jax.experimental.pallas import tpu as pltpu
from jax.experimental.pallas import tpu_sc as plsc

N = 10000
E = 320000
B = 65536
D_IN = 128
D_HID = 256
D_EMB = 128

NC = 2
NS = 16
K = 128

CH = 160
E_PAD = NS * CH * K
N_DUMMY = 10008
N_PAD = 10112
ROWS_I = N_PAD // NS
ROWS_W = 624
ROWS_TAIL = N - NS * ROWS_W

BD = B // NS
CHD = BD // K

@functools.lru_cache(maxsize=None)
def _mesh():
    return plsc.VectorSubcoreMesh(
        core_axis_name="c", subcore_axis_name="s",
        num_cores=NC, num_subcores=NS)


_f32 = jnp.float32
_i32 = jnp.int32


def _mul(v, m):
    return pl.multiple_of(v, m)


NBUF = 2
NIDX = 4
G = NS * CH + 2


def _agg_loop(table_hbm, out_hbm, ei_hbm, ib2, gbuf, acc, isems, gsems,
              ssems, s, base, L):
    def idx_cp(chunk, q):
        return pltpu.make_async_copy(ei_hbm.at[base + chunk],
                                     ib2.at[pl.ds(2 * q, 2)], isems[q])

    def gather(q, b):
        return pltpu.make_async_copy(table_hbm.at[ib2.at[2 * q]],
                                     gbuf.at[b], gsems[b])

    def scat_start(q, b):
        pltpu.async_copy(gbuf.at[b], acc.at[ib2.at[2 * q + 1]], ssems[b],
                         add=True)

    def scat_wait(q, b):
        pltpu.make_async_copy(gbuf.at[b], acc.at[ib2.at[2 * q + 1]],
                              ssems[b]).wait()

    def step(j, u):
        idx_cp(j + 2, (u + 2) % NIDX).start()
        idx_cp(j + 1, (u + 1) % NIDX).wait()
        if u != 0:
            scat_wait((u + 3) % NIDX, (u + 1) % NBUF)
        gather((u + 1) % NIDX, (u + 1) % NBUF).start()
        gather(u, u % NBUF).wait()
        scat_start(u, u % NBUF)

    idx_cp(0, 0).start()
    idx_cp(1, 1).start()
    idx_cp(0, 0).wait()
    gather(0, 0).start()
    for j in range(NIDX):
        step(j, j)

    def body(i, cr):
        for u in range(NIDX):
            j = NIDX * i + NIDX + u
            if u == 0:
                scat_wait(3, 1)
            step(j, u)
        return cr

    lax.fori_loop(0, L // NIDX - 1, body, 0)
    scat_wait(3, 1)
    gather(0, 0).wait()
    idx_cp(L + 1, 1).wait()
    plsc.subcore_barrier()
    _writeback(acc, out_hbm, s)


def _writeback(acc, out_hbm, s):
    off = _mul(s * ROWS_W, 8)
    pltpu.sync_copy(acc.at[pl.ds(off, ROWS_W)], out_hbm.at[pl.ds(off, ROWS_W)])

    @pl.when(s == 0)
    def _():
        pltpu.sync_copy(acc.at[pl.ds(NS * ROWS_W, ROWS_TAIL)],
                        out_hbm.at[pl.ds(NS * ROWS_W, ROWS_TAIL)])


def _zero_init(z_hbm, acc, s):
    off = _mul(s * ROWS_I, 8)
    pltpu.sync_copy(z_hbm.at[pl.ds(off, ROWS_I)], acc.at[pl.ds(off, ROWS_I)])


@functools.lru_cache(maxsize=None)
def _sc_agg_h():
    return pl.kernel(
        _sc_agg_h_body,
        out_type=(jax.ShapeDtypeStruct((N, 128), _f32),
                  jax.ShapeDtypeStruct((N, 128), _f32)),
        mesh=_mesh(),
        scratch_types=[
            pltpu.VMEM((2 * NIDX, K), _i32),
            pltpu.VMEM((NBUF, K, 128), _f32),
            pltpu.VMEM_SHARED((N_PAD, 128), _f32),
            pltpu.SemaphoreType.DMA,
            pltpu.SemaphoreType.DMA,
            pltpu.SemaphoreType.DMA,
            pltpu.SemaphoreType.DMA,
            pltpu.SemaphoreType.DMA,
            pltpu.SemaphoreType.DMA,
            pltpu.SemaphoreType.DMA,
            pltpu.SemaphoreType.DMA,
        ],
    )


def _sc_agg_h_body(ha_hbm, hb_hbm, ei_hbm, z128_hbm,
                   agg_a_out, agg_b_out, ib2, gbuf, acc,
                   is0, is1, is2, is3, gs0, gs1, ss0, ss1):
    c = lax.axis_index("c")
    s = lax.axis_index("s")
    _zero_init(z128_hbm, acc, s)
    plsc.subcore_barrier()
    isems = (is0, is1, is2, is3)
    gsems = (gs0, gs1)
    ssems = (ss0, ss1)

    @pl.when(c == 0)
    def _():
        _agg_loop(ha_hbm, agg_a_out, ei_hbm, ib2, gbuf, acc, isems, gsems,
                  ssems, s, s * CH, CH)

    @pl.when(c == 1)
    def _():
        _agg_loop(hb_hbm, agg_b_out, ei_hbm, ib2, gbuf, acc, isems, gsems,
                  ssems, s, s * CH, CH)


def _deg_loop(out_hbm, ei_hbm, ib2, onesb, acc, isems, ssems, s, base, L):
    def idx_cp(chunk, q):
        return pltpu.make_async_copy(ei_hbm.at[base + chunk],
                                     ib2.at[pl.ds(2 * q, 2)], isems[q])

    def sc_start(q, b):
        pltpu.async_copy(onesb, acc.at[ib2.at[2 * q + 1]], ssems[b], add=True)

    def sc_wait(q, b):
        pltpu.make_async_copy(onesb, acc.at[ib2.at[2 * q + 1]],
                              ssems[b]).wait()

    idx_cp(0, 0).start()
    idx_cp(1, 1).start()

    def step(j, u, warm):
        if warm:
            sc_wait((u + 2) % NIDX, u % NBUF)
        idx_cp(j + 2, (u + 2) % NIDX).start()
        idx_cp(j, u).wait()
        sc_start(u, u % NBUF)

    for j in range(NIDX):
        step(j, j, j >= 2)

    def body(i, cr):
        for u in range(NIDX):
            step(NIDX * i + NIDX + u, u, True)
        return cr

    lax.fori_loop(0, L // NIDX - 1, body, 0)
    sc_wait(2, 0)
    sc_wait(3, 1)
    idx_cp(L, 0).wait()
    idx_cp(L + 1, 1).wait()
    plsc.subcore_barrier()
    _writeback(acc, out_hbm, s)


HC = CH // 2


@functools.lru_cache(maxsize=None)
def _sc_l0():
    return pl.kernel(
        _sc_l0_body,
        out_type=(jax.ShapeDtypeStruct((N, 128), _f32),
                  jax.ShapeDtypeStruct((N, 128), _f32),
                  jax.ShapeDtypeStruct((N, 128), _f32),
                  jax.ShapeDtypeStruct((N, 128), _f32)),
        mesh=_mesh(),
        scratch_types=[
            pltpu.VMEM((2 * NIDX, K), _i32),
            pltpu.VMEM((NBUF, K, 128), _f32),
            pltpu.VMEM_SHARED((N_PAD, 128), _f32),
            pltpu.SemaphoreType.DMA,
            pltpu.SemaphoreType.DMA,
            pltpu.SemaphoreType.DMA,
            pltpu.SemaphoreType.DMA,
            pltpu.SemaphoreType.DMA,
            pltpu.SemaphoreType.DMA,
            pltpu.SemaphoreType.DMA,
            pltpu.SemaphoreType.DMA,
        ],
    )


def _sc_l0_body(x_hbm, ones_hbm, ei_hbm, z128_hbm,
                apa_out, apb_out, dpa_out, dpb_out, ib2, gbuf, acc,
                is0, is1, is2, is3, gs0, gs1, ss0, ss1):
    c = lax.axis_index("c")
    s = lax.axis_index("s")
    isems = (is0, is1, is2, is3)
    gsems = (gs0, gs1)
    ssems = (ss0, ss1)
    base = (c * NS + s) * HC
    _zero_init(z128_hbm, acc, s)
    plsc.subcore_barrier()

    @pl.when(c == 0)
    def _():
        _agg_loop(x_hbm, apa_out, ei_hbm, ib2, gbuf, acc, isems, gsems,
                  ssems, s, base, HC)

    @pl.when(c == 1)
    def _():
        _agg_loop(x_hbm, apb_out, ei_hbm, ib2, gbuf, acc, isems, gsems,
                  ssems, s, base, HC)

    plsc.subcore_barrier()
    pltpu.sync_copy(ones_hbm.at[pl.ds(0, K)], gbuf.at[0])
    _zero_init(z128_hbm, acc, s)
    plsc.subcore_barrier()

    @pl.when(c == 0)
    def _():
        _deg_loop(dpa_out, ei_hbm, ib2, gbuf.at[0], acc, isems, ssems, s,
                  base, HC)

    @pl.when(c == 1)
    def _():
        _deg_loop(dpb_out, ei_hbm, ib2, gbuf.at[0], acc, isems, ssems, s,
                  base, HC)


@functools.lru_cache(maxsize=None)
def _sc_agg_split():
    return pl.kernel(
        _sc_agg_split_body,
        out_type=(jax.ShapeDtypeStruct((N, 128), _f32),
                  jax.ShapeDtypeStruct((N, 128), _f32)),
        mesh=_mesh(),
        scratch_types=[
            pltpu.VMEM((2 * NIDX, K), _i32),
            pltpu.VMEM((NBUF, K, 128), _f32),
            pltpu.VMEM_SHARED((N_PAD, 128), _f32),
            pltpu.SemaphoreType.DMA,
            pltpu.SemaphoreType.DMA,
            pltpu.SemaphoreType.DMA,
            pltpu.SemaphoreType.DMA,
            pltpu.SemaphoreType.DMA,
            pltpu.SemaphoreType.DMA,
            pltpu.SemaphoreType.DMA,
            pltpu.SemaphoreType.DMA,
        ],
    )


def _sc_agg_split_body(tab_hbm, ei_hbm, z128_hbm, pa_out, pb_out,
                       ib2, gbuf, acc, is0, is1, is2, is3, gs0, gs1, ss0,
                       ss1):
    c = lax.axis_index("c")
    s = lax.axis_index("s")
    isems = (is0, is1, is2, is3)
    gsems = (gs0, gs1)
    ssems = (ss0, ss1)
    base = (c * NS + s) * HC
    _zero_init(z128_hbm, acc, s)
    plsc.subcore_barrier()

    @pl.when(c == 0)
    def _():
        _agg_loop(tab_hbm, pa_out, ei_hbm, ib2, gbuf, acc, isems, gsems,
                  ssems, s, base, HC)

    @pl.when(c == 1)
    def _():
        _agg_loop(tab_hbm, pb_out, ei_hbm, ib2, gbuf, acc, isems, gsems,
                  ssems, s, base, HC)


@functools.lru_cache(maxsize=None)
def _sc_decoder_gather():
    return pl.kernel(
        _sc_decoder_gather_body,
        out_type=(jax.ShapeDtypeStruct((B, 128), _f32),
                  jax.ShapeDtypeStruct((B, 128), _f32)),
        mesh=_mesh(),
        scratch_types=[
            pltpu.VMEM((NBUF, K), _i32),
            pltpu.VMEM((NBUF, K, 128), _f32),
            pltpu.SemaphoreType.DMA,
            pltpu.SemaphoreType.DMA,
        ],
    )


def _sc_decoder_gather_body(u_hbm, z_hbm, eli_hbm, gs_out, gd_out, idx, gbuf,
                            sem0, sem1):
    c = lax.axis_index("c")
    s = lax.axis_index("s")
    sems = (sem0, sem1)

    def gather_to(table_hbm, out_hbm):
        def ld_idx(chunk, b):
            off = _mul(c * B + (s * CHD + chunk) * K, K)
            pltpu.sync_copy(eli_hbm.at[pl.ds(off, K)], idx.at[b])

        def gather(b):
            return pltpu.make_async_copy(table_hbm.at[idx.at[b]], gbuf.at[b],
                                         sems[b])

        for b in range(NBUF):
            ld_idx(b, b)
            gather(b).start()

        def body(i, cr):
            for b in range(NBUF):
                j = i * NBUF + b
                gather(b).wait()
                row = _mul(s * BD + j * K, K)
                pltpu.sync_copy(gbuf.at[b], out_hbm.at[pl.ds(row, K)])
                ld_idx(j + NBUF, b)
                gather(b).start()
            return cr

        lax.fori_loop(0, CHD // NBUF, body, 0)
        for b in range(NBUF):
            gather(b).wait()

    @pl.when(c == 0)
    def _():
        gather_to(u_hbm, gs_out)

    @pl.when(c == 1)
    def _():
        gather_to(z_hbm, gd_out)


BLK = 1000
GRID = N // BLK

_DOT = functools.partial(lax.dot_general,
                         dimension_numbers=(((1,), (0,)), ((), ())),
                         precision=lax.Precision.HIGHEST,
                         preferred_element_type=_f32)


def _mm_stats_body(n_parts, deg_ref, *refs):
    a_refs = refs[:n_parts]
    x_refs = refs[n_parts:2 * n_parts]
    wl_ref, wr_ref, b_ref, out_ref, stats_ref = refs[2 * n_parts:]
    i = pl.program_id(0)
    deg = jnp.maximum(deg_ref[:, 0:1], 1.0)
    hp = b_ref[...]
    for p in range(n_parts):
        hp = hp + _DOT(a_refs[p][...] / deg, wl_ref[pl.ds(p * 128, 128), :])
        hp = hp + _DOT(x_refs[p][...], wr_ref[pl.ds(p * 128, 128), :])
    out_ref[...] = hp

    @pl.when(i == 0)
    def _():
        stats_ref[...] = jnp.zeros_like(stats_ref)
    stats_ref[0:1, :] += jnp.sum(hp, axis=0, keepdims=True)
    stats_ref[1:2, :] += jnp.sum(hp * hp, axis=0, keepdims=True)


def _tc_mm_stats(n_parts, aggs, xs, deg16, W_l, W_r, b):
    d_out = W_l.shape[1]
    in_specs = (
        [pl.BlockSpec((BLK, 16), lambda i: (i, 0))]
        + [pl.BlockSpec((BLK, 128), lambda i: (i, 0))] * (2 * n_parts)
        + [pl.BlockSpec(W_l.shape, lambda i: (0, 0)),
           pl.BlockSpec(W_r.shape, lambda i: (0, 0)),
           pl.BlockSpec((1, d_out), lambda i: (0, 0))]
    )
    return pl.pallas_call(
        functools.partial(_mm_stats_body, n_parts),
        grid=(GRID,),
        in_specs=in_specs,
        out_specs=(pl.BlockSpec((BLK, d_out), lambda i: (i, 0)),
                   pl.BlockSpec((2, d_out), lambda i: (0, 0))),
        out_shape=(jax.ShapeDtypeStruct((N, d_out), _f32),
                   jax.ShapeDtypeStruct((2, d_out), _f32)),
    )(deg16, *aggs, *xs, W_l, W_r, b.reshape(1, d_out))


def _l0_fused_body(dpa_ref, dpb_ref, pa_ref, pb_ref, x_ref, wl_ref, wr_ref,
                   b_ref, g_ref, be_ref, ha_ref, hb_ref, deg_ref,
                   hp_s, st_s):
    i = pl.program_id(0)
    deg16 = dpa_ref[...] + dpb_ref[...]
    deg_ref[...] = deg16

    @pl.when(i < GRID)
    def _():
        deg = jnp.maximum(deg16[:, 0:1], 1.0)
        agg = (pa_ref[...] + pb_ref[...]) / deg
        hp = b_ref[...] + _DOT(agg, wl_ref[...]) + _DOT(x_ref[...], wr_ref[...])
        hp_s[pl.ds(i * BLK, BLK), :] = hp

        @pl.when(i == 0)
        def _():
            st_s[...] = jnp.zeros_like(st_s)
        st_s[0:1, :] += jnp.sum(hp, axis=0, keepdims=True)
        st_s[1:2, :] += jnp.sum(hp * hp, axis=0, keepdims=True)

    @pl.when(i >= GRID)
    def _():
        hp = hp_s[pl.ds((i - GRID) * BLK, BLK), :]
        inv_n = _f32(1.0 / N)
        mean = st_s[0:1, :] * inv_n
        var = st_s[1:2, :] * inv_n - mean * mean
        inv = lax.rsqrt(var + 1e-5)
        h = jnp.maximum((hp - mean) * inv * g_ref[...] + be_ref[...], 0.0)
        ha_ref[...] = h[:, 0:128]
        hb_ref[...] = h[:, 128:256]


def _tc_layer0(dpa16, dpb16, pa, pb, x, W_l, W_r, b, g, be):
    blk = lambda: pl.BlockSpec((BLK, 128), lambda i: (i % GRID, 0))
    row = lambda: pl.BlockSpec((1, D_HID), lambda i: (0, 0))
    return pl.pallas_call(
        _l0_fused_body,
        grid=(2 * GRID,),
        in_specs=[pl.BlockSpec((BLK, 16), lambda i: (i % GRID, 0)),
                  pl.BlockSpec((BLK, 16), lambda i: (i % GRID, 0)),
                  blk(), blk(), blk(),
                  pl.BlockSpec((D_IN, D_HID), lambda i: (0, 0)),
                  pl.BlockSpec((D_IN, D_HID), lambda i: (0, 0)),
                  row(), row(), row()],
        out_specs=(pl.BlockSpec((BLK, 128), lambda i: (i % GRID, 0)),
                   pl.BlockSpec((BLK, 128), lambda i: (i % GRID, 0)),
                   pl.BlockSpec((BLK, 16), lambda i: (i % GRID, 0))),
        out_shape=(jax.ShapeDtypeStruct((N, 128), _f32),
                   jax.ShapeDtypeStruct((N, 128), _f32),
                   jax.ShapeDtypeStruct((N, 16), _f32)),
        scratch_shapes=[pltpu.VMEM((N, D_HID), _f32),
                        pltpu.VMEM((2, D_HID), _f32)],
    )(dpa16, dpb16, pa, pb, x, W_l, W_r, b.reshape(1, D_HID),
      g.reshape(1, D_HID), be.reshape(1, D_HID))


def _l1_fused_body(deg_ref, a0_ref, a1_ref, x0_ref, x1_ref, wl_ref, wr_ref,
                   b_ref, g_ref, be_ref, wl2_ref, ha_ref, hb_ref, y_ref,
                   hp_s, st_s):
    i = pl.program_id(0)

    @pl.when(i < GRID)
    def _():
        deg = jnp.maximum(deg_ref[:, 0:1], 1.0)
        hp = b_ref[...]
        hp = hp + _DOT(a0_ref[...] / deg, wl_ref[0:128, :])
        hp = hp + _DOT(a1_ref[...] / deg, wl_ref[128:256, :])
        hp = hp + _DOT(x0_ref[...], wr_ref[0:128, :])
        hp = hp + _DOT(x1_ref[...], wr_ref[128:256, :])
        hp_s[pl.ds(i * BLK, BLK), :] = hp

        @pl.when(i == 0)
        def _():
            st_s[...] = jnp.zeros_like(st_s)
        st_s[0:1, :] += jnp.sum(hp, axis=0, keepdims=True)
        st_s[1:2, :] += jnp.sum(hp * hp, axis=0, keepdims=True)

    @pl.when(i >= GRID)
    def _():
        hp = hp_s[pl.ds((i - GRID) * BLK, BLK), :]
        inv_n = _f32(1.0 / N)
        mean = st_s[0:1, :] * inv_n
        var = st_s[1:2, :] * inv_n - mean * mean
        inv = lax.rsqrt(var + 1e-5)
        h = jnp.maximum((hp - mean) * inv * g_ref[...] + be_ref[...], 0.0)
        ha_ref[...] = h[:, 0:128]
        hb_ref[...] = h[:, 128:256]
        y_ref[...] = _DOT(h, wl2_ref[...])


def _tc_layer1(deg16, aggs, xs, W_l, W_r, b, g, be, W_l2):
    blk = lambda: pl.BlockSpec((BLK, 128), lambda i: (i % GRID, 0))
    row = lambda: pl.BlockSpec((1, D_HID), lambda i: (0, 0))
    wmat = lambda: pl.BlockSpec((D_HID, D_HID), lambda i: (0, 0))
    return pl.pallas_call(
        _l1_fused_body,
        grid=(2 * GRID,),
        in_specs=[pl.BlockSpec((BLK, 16), lambda i: (i % GRID, 0)),
                  blk(), blk(), blk(), blk(),
                  wmat(), wmat(), row(), row(), row(),
                  pl.BlockSpec((D_HID, D_EMB), lambda i: (0, 0))],
        out_specs=(pl.BlockSpec((BLK, 128), lambda i: (i % GRID, 0)),
                   pl.BlockSpec((BLK, 128), lambda i: (i % GRID, 0)),
                   pl.BlockSpec((BLK, D_EMB), lambda i: (i % GRID, 0))),
        out_shape=(jax.ShapeDtypeStruct((N, 128), _f32),
                   jax.ShapeDtypeStruct((N, 128), _f32),
                   jax.ShapeDtypeStruct((N, D_EMB), _f32)),
        scratch_shapes=[pltpu.VMEM((N, D_HID), _f32),
                        pltpu.VMEM((2, D_HID), _f32)],
    )(deg16, *aggs, *xs, W_l, W_r, b.reshape(1, D_HID),
      g.reshape(1, D_HID), be.reshape(1, D_HID), W_l2)


def _mm_final_body(deg_ref, pa_ref, pb_ref, x0_ref, x1_ref,
                   wr_ref, b_ref, wbi_ref, z_ref, u_ref):
    deg = jnp.maximum(deg_ref[:, 0:1], 1.0)
    z = b_ref[...] + (pa_ref[...] + pb_ref[...]) / deg
    z = z + _DOT(x0_ref[...], wr_ref[0:128, :])
    z = z + _DOT(x1_ref[...], wr_ref[128:256, :])
    z_ref[...] = z
    u_ref[...] = _DOT(z, wbi_ref[...])


def _tc_mm_final(aggs, xs, deg16, W_r, b, W_bi0):
    return pl.pallas_call(
        _mm_final_body,
        grid=(GRID,),
        in_specs=[pl.BlockSpec((BLK, 16), lambda i: (i, 0))]
        + [pl.BlockSpec((BLK, 128), lambda i: (i, 0))] * 4
        + [pl.BlockSpec((D_HID, D_EMB), lambda i: (0, 0)),
           pl.BlockSpec((1, D_EMB), lambda i: (0, 0)),
           pl.BlockSpec((D_EMB, D_EMB), lambda i: (0, 0))],
        out_specs=(pl.BlockSpec((BLK, D_EMB), lambda i: (i, 0)),
                   pl.BlockSpec((BLK, D_EMB), lambda i: (i, 0))),
        out_shape=(jax.ShapeDtypeStruct((N, D_EMB), _f32),
                   jax.ShapeDtypeStruct((N, D_EMB), _f32)),
    )(deg16, *aggs, *xs, W_r, b.reshape(1, D_EMB), W_bi0)


DBLK = 8192
DGRID = B // DBLK


def _dot_body(gs_ref, gd_ref, b_ref, out_ref):
    r = jnp.sum(gs_ref[...] * gd_ref[...], axis=1)
    out_ref[...] = r.reshape(DBLK // 128, 128) + b_ref[0, 0]


def _tc_decoder_dot(gs, gd, b_bi):
    out2d = pl.pallas_call(
        _dot_body,
        grid=(DGRID,),
        in_specs=[pl.BlockSpec((DBLK, 128), lambda i: (i, 0)),
                  pl.BlockSpec((DBLK, 128), lambda i: (i, 0)),
                  pl.BlockSpec((1, 1), lambda i: (0, 0))],
        out_specs=pl.BlockSpec((DBLK // 128, 128), lambda i: (i, 0)),
        out_shape=jax.ShapeDtypeStruct((B // 128, 128), _f32),
    )(gs, gd, b_bi.reshape(1, 1))
    return out2d.reshape(B)


def kernel(x, edge_index, edge_label_index, W_l0, W_r0, b0, W_l1, W_r1, b1,
           W_l2, W_r2, b2, g1, be1, g2, be2, W_bi, b_bi):
    src = edge_index[0].astype(_i32)
    dst = edge_index[1].astype(_i32)
    pad = G * K - E
    srcp = jnp.concatenate([src, jnp.zeros((pad,), _i32)]).reshape(G, 1, K)
    dstp = jnp.concatenate([dst, jnp.full((pad,), N_DUMMY, _i32)]).reshape(G, 1, K)
    ei3 = jnp.concatenate([srcp, dstp], axis=1)
    eli = jnp.concatenate([edge_label_index.astype(_i32).reshape(2 * B),
                           jnp.zeros((NBUF * K,), _i32)])
    z128 = jnp.zeros((N_PAD, 128), _f32)

    ones128 = jnp.ones((N, 128), _f32)
    apa, apb, dpa, dpb = _sc_l0()(x, ones128, ei3, z128)
    h1a, h1b, deg16 = _tc_layer0(dpa[:, :16], dpb[:, :16], apa, apb, x,
                                 W_l0, W_r0, b0, g1, be1)

    agg1a, agg1b = _sc_agg_h()(h1a, h1b, ei3, z128)
    h2a, h2b, y2 = _tc_layer1(deg16, (agg1a, agg1b), (h1a, h1b),
                              W_l1, W_r1, b1, g2, be2, W_l2)

    p2a, p2b = _sc_agg_split()(y2, ei3, z128)
    z, u = _tc_mm_final((p2a, p2b), (h2a, h2b), deg16, W_r2, b2, W_bi[0])

    gs, gd = _sc_decoder_gather()(u, z, eli)
    return _tc_decoder_dot(gs, gd, b_bi)


# speedup vs baseline: 1.0305x; 1.0305x over previous
import functools

import jax
import jax.numpy as jnp
from jax import lax
from jax.experimental import pallas as pl
from jax.experimental.pallas import tpu as pltpu
from jax.experimental.pallas import tpu_sc as plsc

N = 10000
E = 320000
B = 65536
D_IN = 128
D_HID = 256
D_EMB = 128

NC = 2
NS = 16
K = 128

CH = 160
E_PAD = NS * CH * K
N_DUMMY = 10008
N_PAD = 10112
ROWS_I = N_PAD // NS
ROWS_W = 624
ROWS_TAIL = N - NS * ROWS_W

BD = B // NS
CHD = BD // K

@functools.lru_cache(maxsize=None)
def _mesh():
    return plsc.VectorSubcoreMesh(
        core_axis_name="c", subcore_axis_name="s",
        num_cores=NC, num_subcores=NS)


_f32 = jnp.float32
_i32 = jnp.int32


def _mul(v, m):
    return pl.multiple_of(v, m)


NBUF = 2
NIDX = 4
G = NS * CH + 2


def _agg_loop(table_hbm, out_hbm, ei_hbm, ib2, gbuf, acc, isems, gsems,
              ssems, s, base, L):
    def idx_cp(chunk, q):
        return pltpu.make_async_copy(ei_hbm.at[base + chunk],
                                     ib2.at[pl.ds(2 * q, 2)], isems[q])

    def gather(q, b):
        return pltpu.make_async_copy(table_hbm.at[ib2.at[2 * q]],
                                     gbuf.at[b], gsems[b])

    def scat_start(q, b):
        pltpu.async_copy(gbuf.at[b], acc.at[ib2.at[2 * q + 1]], ssems[b],
                         add=True)

    def scat_wait(q, b):
        pltpu.make_async_copy(gbuf.at[b], acc.at[ib2.at[2 * q + 1]],
                              ssems[b]).wait()

    def step(j, u):
        idx_cp(j + 2, (u + 2) % NIDX).start()
        idx_cp(j + 1, (u + 1) % NIDX).wait()
        if u != 0:
            scat_wait((u + 3) % NIDX, (u + 1) % NBUF)
        gather((u + 1) % NIDX, (u + 1) % NBUF).start()
        gather(u, u % NBUF).wait()
        scat_start(u, u % NBUF)

    idx_cp(0, 0).start()
    idx_cp(1, 1).start()
    idx_cp(0, 0).wait()
    gather(0, 0).start()
    for j in range(NIDX):
        step(j, j)

    def body(i, cr):
        for u in range(NIDX):
            j = NIDX * i + NIDX + u
            if u == 0:
                scat_wait(3, 1)
            step(j, u)
        return cr

    lax.fori_loop(0, L // NIDX - 1, body, 0)
    scat_wait(3, 1)
    gather(0, 0).wait()
    idx_cp(L + 1, 1).wait()
    plsc.subcore_barrier()
    _writeback(acc, out_hbm, s)


def _writeback(acc, out_hbm, s):
    off = _mul(s * ROWS_W, 8)
    pltpu.sync_copy(acc.at[pl.ds(off, ROWS_W)], out_hbm.at[pl.ds(off, ROWS_W)])

    @pl.when(s == 0)
    def _():
        pltpu.sync_copy(acc.at[pl.ds(NS * ROWS_W, ROWS_TAIL)],
                        out_hbm.at[pl.ds(NS * ROWS_W, ROWS_TAIL)])


def _zero_init(z_hbm, acc, s):
    off = _mul(s * ROWS_I, 8)
    pltpu.sync_copy(z_hbm.at[pl.ds(off, ROWS_I)], acc.at[pl.ds(off, ROWS_I)])


@functools.lru_cache(maxsize=None)
def _sc_agg_h():
    return pl.kernel(
        _sc_agg_h_body,
        out_type=(jax.ShapeDtypeStruct((N, 128), _f32),
                  jax.ShapeDtypeStruct((N, 128), _f32)),
        mesh=_mesh(),
        scratch_types=[
            pltpu.VMEM((2 * NIDX, K), _i32),
            pltpu.VMEM((NBUF, K, 128), _f32),
            pltpu.VMEM_SHARED((N_PAD, 128), _f32),
            pltpu.SemaphoreType.DMA,
            pltpu.SemaphoreType.DMA,
            pltpu.SemaphoreType.DMA,
            pltpu.SemaphoreType.DMA,
            pltpu.SemaphoreType.DMA,
            pltpu.SemaphoreType.DMA,
            pltpu.SemaphoreType.DMA,
            pltpu.SemaphoreType.DMA,
        ],
    )


def _sc_agg_h_body(ha_hbm, hb_hbm, ei_hbm, z128_hbm,
                   agg_a_out, agg_b_out, ib2, gbuf, acc,
                   is0, is1, is2, is3, gs0, gs1, ss0, ss1):
    c = lax.axis_index("c")
    s = lax.axis_index("s")
    _zero_init(z128_hbm, acc, s)
    plsc.subcore_barrier()
    isems = (is0, is1, is2, is3)
    gsems = (gs0, gs1)
    ssems = (ss0, ss1)

    @pl.when(c == 0)
    def _():
        _agg_loop(ha_hbm, agg_a_out, ei_hbm, ib2, gbuf, acc, isems, gsems,
                  ssems, s, s * CH, CH)

    @pl.when(c == 1)
    def _():
        _agg_loop(hb_hbm, agg_b_out, ei_hbm, ib2, gbuf, acc, isems, gsems,
                  ssems, s, s * CH, CH)


def _deg_loop(out_hbm, ei_hbm, ib2, onesb, acc, isems, ssems, s, base, L):
    def idx_cp(chunk, q):
        return pltpu.make_async_copy(ei_hbm.at[base + chunk],
                                     ib2.at[pl.ds(2 * q, 2)], isems[q])

    def sc_start(q, b):
        pltpu.async_copy(onesb, acc.at[ib2.at[2 * q + 1]], ssems[b], add=True)

    def sc_wait(q, b):
        pltpu.make_async_copy(onesb, acc.at[ib2.at[2 * q + 1]],
                              ssems[b]).wait()

    idx_cp(0, 0).start()
    idx_cp(1, 1).start()

    def step(j, u, warm):
        if warm:
            sc_wait((u + 2) % NIDX, u % NBUF)
        idx_cp(j + 2, (u + 2) % NIDX).start()
        idx_cp(j, u).wait()
        sc_start(u, u % NBUF)

    for j in range(NIDX):
        step(j, j, j >= 2)

    def body(i, cr):
        for u in range(NIDX):
            step(NIDX * i + NIDX + u, u, True)
        return cr

    lax.fori_loop(0, L // NIDX - 1, body, 0)
    sc_wait(2, 0)
    sc_wait(3, 1)
    idx_cp(L, 0).wait()
    idx_cp(L + 1, 1).wait()
    plsc.subcore_barrier()
    _writeback(acc, out_hbm, s)


HC = CH // 2


@functools.lru_cache(maxsize=None)
def _sc_l0():
    return pl.kernel(
        _sc_l0_body,
        out_type=(jax.ShapeDtypeStruct((N, 128), _f32),
                  jax.ShapeDtypeStruct((N, 128), _f32),
                  jax.ShapeDtypeStruct((N, 128), _f32),
                  jax.ShapeDtypeStruct((N, 128), _f32)),
        mesh=_mesh(),
        scratch_types=[
            pltpu.VMEM((2 * NIDX, K), _i32),
            pltpu.VMEM((NBUF, K, 128), _f32),
            pltpu.VMEM_SHARED((N_PAD, 128), _f32),
            pltpu.SemaphoreType.DMA,
            pltpu.SemaphoreType.DMA,
            pltpu.SemaphoreType.DMA,
            pltpu.SemaphoreType.DMA,
            pltpu.SemaphoreType.DMA,
            pltpu.SemaphoreType.DMA,
            pltpu.SemaphoreType.DMA,
            pltpu.SemaphoreType.DMA,
        ],
    )


def _sc_l0_body(x_hbm, ones_hbm, ei_hbm, z128_hbm,
                apa_out, apb_out, dpa_out, dpb_out, ib2, gbuf, acc,
                is0, is1, is2, is3, gs0, gs1, ss0, ss1):
    c = lax.axis_index("c")
    s = lax.axis_index("s")
    isems = (is0, is1, is2, is3)
    gsems = (gs0, gs1)
    ssems = (ss0, ss1)
    base = (c * NS + s) * HC
    _zero_init(z128_hbm, acc, s)
    plsc.subcore_barrier()

    @pl.when(c == 0)
    def _():
        _agg_loop(x_hbm, apa_out, ei_hbm, ib2, gbuf, acc, isems, gsems,
                  ssems, s, base, HC)

    @pl.when(c == 1)
    def _():
        _agg_loop(x_hbm, apb_out, ei_hbm, ib2, gbuf, acc, isems, gsems,
                  ssems, s, base, HC)

    plsc.subcore_barrier()
    pltpu.sync_copy(ones_hbm.at[pl.ds(0, K)], gbuf.at[0])
    _zero_init(z128_hbm, acc, s)
    plsc.subcore_barrier()

    @pl.when(c == 0)
    def _():
        _deg_loop(dpa_out, ei_hbm, ib2, gbuf.at[0], acc, isems, ssems, s,
                  base, HC)

    @pl.when(c == 1)
    def _():
        _deg_loop(dpb_out, ei_hbm, ib2, gbuf.at[0], acc, isems, ssems, s,
                  base, HC)


@functools.lru_cache(maxsize=None)
def _sc_agg_split():
    return pl.kernel(
        _sc_agg_split_body,
        out_type=(jax.ShapeDtypeStruct((N, 128), _f32),
                  jax.ShapeDtypeStruct((N, 128), _f32)),
        mesh=_mesh(),
        scratch_types=[
            pltpu.VMEM((2 * NIDX, K), _i32),
            pltpu.VMEM((NBUF, K, 128), _f32),
            pltpu.VMEM_SHARED((N_PAD, 128), _f32),
            pltpu.SemaphoreType.DMA,
            pltpu.SemaphoreType.DMA,
            pltpu.SemaphoreType.DMA,
            pltpu.SemaphoreType.DMA,
            pltpu.SemaphoreType.DMA,
            pltpu.SemaphoreType.DMA,
            pltpu.SemaphoreType.DMA,
            pltpu.SemaphoreType.DMA,
        ],
    )


def _sc_agg_split_body(tab_hbm, ei_hbm, z128_hbm, pa_out, pb_out,
                       ib2, gbuf, acc, is0, is1, is2, is3, gs0, gs1, ss0,
                       ss1):
    c = lax.axis_index("c")
    s = lax.axis_index("s")
    isems = (is0, is1, is2, is3)
    gsems = (gs0, gs1)
    ssems = (ss0, ss1)
    base = (c * NS + s) * HC
    _zero_init(z128_hbm, acc, s)
    plsc.subcore_barrier()

    @pl.when(c == 0)
    def _():
        _agg_loop(tab_hbm, pa_out, ei_hbm, ib2, gbuf, acc, isems, gsems,
                  ssems, s, base, HC)

    @pl.when(c == 1)
    def _():
        _agg_loop(tab_hbm, pb_out, ei_hbm, ib2, gbuf, acc, isems, gsems,
                  ssems, s, base, HC)


@functools.lru_cache(maxsize=None)
def _sc_decoder_gather():
    return pl.kernel(
        _sc_decoder_gather_body,
        out_type=(jax.ShapeDtypeStruct((B, 128), _f32),
                  jax.ShapeDtypeStruct((B, 128), _f32)),
        mesh=_mesh(),
        scratch_types=[
            pltpu.VMEM((NBUF, K), _i32),
            pltpu.VMEM((NBUF, K, 128), _f32),
            pltpu.SemaphoreType.DMA,
            pltpu.SemaphoreType.DMA,
        ],
    )


def _sc_decoder_gather_body(u_hbm, z_hbm, eli_hbm, gs_out, gd_out, idx, gbuf,
                            sem0, sem1):
    c = lax.axis_index("c")
    s = lax.axis_index("s")
    sems = (sem0, sem1)

    def gather_to(table_hbm, out_hbm):
        def ld_idx(chunk, b):
            off = _mul(c * B + (s * CHD + chunk) * K, K)
            pltpu.sync_copy(eli_hbm.at[pl.ds(off, K)], idx.at[b])

        def gather(b):
            return pltpu.make_async_copy(table_hbm.at[idx.at[b]], gbuf.at[b],
                                         sems[b])

        for b in range(NBUF):
            ld_idx(b, b)
            gather(b).start()

        def body(i, cr):
            for b in range(NBUF):
                j = i * NBUF + b
                gather(b).wait()
                row = _mul(s * BD + j * K, K)
                pltpu.sync_copy(gbuf.at[b], out_hbm.at[pl.ds(row, K)])
                ld_idx(j + NBUF, b)
                gather(b).start()
            return cr

        lax.fori_loop(0, CHD // NBUF, body, 0)
        for b in range(NBUF):
            gather(b).wait()

    @pl.when(c == 0)
    def _():
        gather_to(u_hbm, gs_out)

    @pl.when(c == 1)
    def _():
        gather_to(z_hbm, gd_out)


BLK = 1000
GRID = N // BLK

_DOT = functools.partial(lax.dot_general,
                         dimension_numbers=(((1,), (0,)), ((), ())),
                         precision=lax.Precision.HIGHEST,
                         preferred_element_type=_f32)


def _mm_stats_body(n_parts, deg_ref, *refs):
    a_refs = refs[:n_parts]
    x_refs = refs[n_parts:2 * n_parts]
    wl_ref, wr_ref, b_ref, out_ref, stats_ref = refs[2 * n_parts:]
    i = pl.program_id(0)
    deg = jnp.maximum(deg_ref[:, 0:1], 1.0)
    hp = b_ref[...]
    for p in range(n_parts):
        hp = hp + _DOT(a_refs[p][...] / deg, wl_ref[pl.ds(p * 128, 128), :])
        hp = hp + _DOT(x_refs[p][...], wr_ref[pl.ds(p * 128, 128), :])
    out_ref[...] = hp

    @pl.when(i == 0)
    def _():
        stats_ref[...] = jnp.zeros_like(stats_ref)
    stats_ref[0:1, :] += jnp.sum(hp, axis=0, keepdims=True)
    stats_ref[1:2, :] += jnp.sum(hp * hp, axis=0, keepdims=True)


def _tc_mm_stats(n_parts, aggs, xs, deg16, W_l, W_r, b):
    d_out = W_l.shape[1]
    in_specs = (
        [pl.BlockSpec((BLK, 16), lambda i: (i, 0))]
        + [pl.BlockSpec((BLK, 128), lambda i: (i, 0))] * (2 * n_parts)
        + [pl.BlockSpec(W_l.shape, lambda i: (0, 0)),
           pl.BlockSpec(W_r.shape, lambda i: (0, 0)),
           pl.BlockSpec((1, d_out), lambda i: (0, 0))]
    )
    return pl.pallas_call(
        functools.partial(_mm_stats_body, n_parts),
        grid=(GRID,),
        in_specs=in_specs,
        out_specs=(pl.BlockSpec((BLK, d_out), lambda i: (i, 0)),
                   pl.BlockSpec((2, d_out), lambda i: (0, 0))),
        out_shape=(jax.ShapeDtypeStruct((N, d_out), _f32),
                   jax.ShapeDtypeStruct((2, d_out), _f32)),
    )(deg16, *aggs, *xs, W_l, W_r, b.reshape(1, d_out))


def _mm0_body(dpa_ref, dpb_ref, pa_ref, pb_ref, x_ref, wl_ref, wr_ref, b_ref,
              out_ref, stats_ref, deg_ref):
    i = pl.program_id(0)
    deg16 = dpa_ref[...] + dpb_ref[...]
    deg_ref[...] = deg16
    deg = jnp.maximum(deg16[:, 0:1], 1.0)
    agg = (pa_ref[...] + pb_ref[...]) / deg
    hp = b_ref[...] + _DOT(agg, wl_ref[...]) + _DOT(x_ref[...], wr_ref[...])
    out_ref[...] = hp

    @pl.when(i == 0)
    def _():
        stats_ref[...] = jnp.zeros_like(stats_ref)
    stats_ref[0:1, :] += jnp.sum(hp, axis=0, keepdims=True)
    stats_ref[1:2, :] += jnp.sum(hp * hp, axis=0, keepdims=True)


def _tc_mm0(dpa16, dpb16, pa, pb, x, W_l, W_r, b):
    d_out = W_l.shape[1]
    return pl.pallas_call(
        _mm0_body,
        grid=(GRID,),
        in_specs=[pl.BlockSpec((BLK, 16), lambda i: (i, 0)),
                  pl.BlockSpec((BLK, 16), lambda i: (i, 0)),
                  pl.BlockSpec((BLK, 128), lambda i: (i, 0)),
                  pl.BlockSpec((BLK, 128), lambda i: (i, 0)),
                  pl.BlockSpec((BLK, 128), lambda i: (i, 0)),
                  pl.BlockSpec(W_l.shape, lambda i: (0, 0)),
                  pl.BlockSpec(W_r.shape, lambda i: (0, 0)),
                  pl.BlockSpec((1, d_out), lambda i: (0, 0))],
        out_specs=(pl.BlockSpec((BLK, d_out), lambda i: (i, 0)),
                   pl.BlockSpec((2, d_out), lambda i: (0, 0)),
                   pl.BlockSpec((BLK, 16), lambda i: (i, 0))),
        out_shape=(jax.ShapeDtypeStruct((N, d_out), _f32),
                   jax.ShapeDtypeStruct((2, d_out), _f32),
                   jax.ShapeDtypeStruct((N, 16), _f32)),
    )(dpa16, dpb16, pa, pb, x, W_l, W_r, b.reshape(1, d_out))


def _norm_body(hp_ref, stats_ref, g_ref, be_ref, ha_ref, hb_ref):
    inv_n = _f32(1.0 / N)
    mean = stats_ref[0:1, :] * inv_n
    var = stats_ref[1:2, :] * inv_n - mean * mean
    inv = lax.rsqrt(var + 1e-5)
    h = jnp.maximum((hp_ref[...] - mean) * inv * g_ref[...] + be_ref[...], 0.0)
    ha_ref[...] = h[:, 0:128]
    hb_ref[...] = h[:, 128:256]


def _tc_norm(hp, stats, g, be):
    return pl.pallas_call(
        _norm_body,
        grid=(GRID,),
        in_specs=[pl.BlockSpec((BLK, D_HID), lambda i: (i, 0)),
                  pl.BlockSpec((2, D_HID), lambda i: (0, 0)),
                  pl.BlockSpec((1, D_HID), lambda i: (0, 0)),
                  pl.BlockSpec((1, D_HID), lambda i: (0, 0))],
        out_specs=(pl.BlockSpec((BLK, 128), lambda i: (i, 0)),
                   pl.BlockSpec((BLK, 128), lambda i: (i, 0))),
        out_shape=(jax.ShapeDtypeStruct((N, 128), _f32),
                   jax.ShapeDtypeStruct((N, 128), _f32)),
    )(hp, stats, g.reshape(1, D_HID), be.reshape(1, D_HID))


def _norm2_body(hp_ref, stats_ref, g_ref, be_ref, wl2_ref,
                ha_ref, hb_ref, y_ref):
    inv_n = _f32(1.0 / N)
    mean = stats_ref[0:1, :] * inv_n
    var = stats_ref[1:2, :] * inv_n - mean * mean
    inv = lax.rsqrt(var + 1e-5)
    h = jnp.maximum((hp_ref[...] - mean) * inv * g_ref[...] + be_ref[...], 0.0)
    ha_ref[...] = h[:, 0:128]
    hb_ref[...] = h[:, 128:256]
    y_ref[...] = _DOT(h, wl2_ref[...])


def _tc_norm2(hp, stats, g, be, W_l2):
    return pl.pallas_call(
        _norm2_body,
        grid=(GRID,),
        in_specs=[pl.BlockSpec((BLK, D_HID), lambda i: (i, 0)),
                  pl.BlockSpec((2, D_HID), lambda i: (0, 0)),
                  pl.BlockSpec((1, D_HID), lambda i: (0, 0)),
                  pl.BlockSpec((1, D_HID), lambda i: (0, 0)),
                  pl.BlockSpec((D_HID, D_EMB), lambda i: (0, 0))],
        out_specs=(pl.BlockSpec((BLK, 128), lambda i: (i, 0)),
                   pl.BlockSpec((BLK, 128), lambda i: (i, 0)),
                   pl.BlockSpec((BLK, D_EMB), lambda i: (i, 0))),
        out_shape=(jax.ShapeDtypeStruct((N, 128), _f32),
                   jax.ShapeDtypeStruct((N, 128), _f32),
                   jax.ShapeDtypeStruct((N, D_EMB), _f32)),
    )(hp, stats, g.reshape(1, D_HID), be.reshape(1, D_HID), W_l2)


def _mm_final_body(deg_ref, pa_ref, pb_ref, x0_ref, x1_ref,
                   wr_ref, b_ref, wbi_ref, z_ref, u_ref):
    deg = jnp.maximum(deg_ref[:, 0:1], 1.0)
    z = b_ref[...] + (pa_ref[...] + pb_ref[...]) / deg
    z = z + _DOT(x0_ref[...], wr_ref[0:128, :])
    z = z + _DOT(x1_ref[...], wr_ref[128:256, :])
    z_ref[...] = z
    u_ref[...] = _DOT(z, wbi_ref[...])


def _tc_mm_final(aggs, xs, deg16, W_r, b, W_bi0):
    return pl.pallas_call(
        _mm_final_body,
        grid=(GRID,),
        in_specs=[pl.BlockSpec((BLK, 16), lambda i: (i, 0))]
        + [pl.BlockSpec((BLK, 128), lambda i: (i, 0))] * 4
        + [pl.BlockSpec((D_HID, D_EMB), lambda i: (0, 0)),
           pl.BlockSpec((1, D_EMB), lambda i: (0, 0)),
           pl.BlockSpec((D_EMB, D_EMB), lambda i: (0, 0))],
        out_specs=(pl.BlockSpec((BLK, D_EMB), lambda i: (i, 0)),
                   pl.BlockSpec((BLK, D_EMB), lambda i: (i, 0))),
        out_shape=(jax.ShapeDtypeStruct((N, D_EMB), _f32),
                   jax.ShapeDtypeStruct((N, D_EMB), _f32)),
    )(deg16, *aggs, *xs, W_r, b.reshape(1, D_EMB), W_bi0)


DBLK = 8192
DGRID = B // DBLK


def _dot_body(gs_ref, gd_ref, b_ref, out_ref):
    r = jnp.sum(gs_ref[...] * gd_ref[...], axis=1)
    out_ref[...] = r.reshape(DBLK // 128, 128) + b_ref[0, 0]


def _tc_decoder_dot(gs, gd, b_bi):
    out2d = pl.pallas_call(
        _dot_body,
        grid=(DGRID,),
        in_specs=[pl.BlockSpec((DBLK, 128), lambda i: (i, 0)),
                  pl.BlockSpec((DBLK, 128), lambda i: (i, 0)),
                  pl.BlockSpec((1, 1), lambda i: (0, 0))],
        out_specs=pl.BlockSpec((DBLK // 128, 128), lambda i: (i, 0)),
        out_shape=jax.ShapeDtypeStruct((B // 128, 128), _f32),
    )(gs, gd, b_bi.reshape(1, 1))
    return out2d.reshape(B)


def kernel(x, edge_index, edge_label_index, W_l0, W_r0, b0, W_l1, W_r1, b1,
           W_l2, W_r2, b2, g1, be1, g2, be2, W_bi, b_bi):
    src = edge_index[0].astype(_i32)
    dst = edge_index[1].astype(_i32)
    pad = G * K - E
    srcp = jnp.concatenate([src, jnp.zeros((pad,), _i32)]).reshape(G, 1, K)
    dstp = jnp.concatenate([dst, jnp.full((pad,), N_DUMMY, _i32)]).reshape(G, 1, K)
    ei3 = jnp.concatenate([srcp, dstp], axis=1)
    eli = jnp.concatenate([edge_label_index.astype(_i32).reshape(2 * B),
                           jnp.zeros((NBUF * K,), _i32)])
    z128 = jnp.zeros((N_PAD, 128), _f32)

    ones128 = jnp.ones((N, 128), _f32)
    apa, apb, dpa, dpb = _sc_l0()(x, ones128, ei3, z128)
    hp1, st1, deg16 = _tc_mm0(dpa[:, :16], dpb[:, :16], apa, apb, x,
                              W_l0, W_r0, b0)
    h1a, h1b = _tc_norm(hp1, st1, g1, be1)

    agg1a, agg1b = _sc_agg_h()(h1a, h1b, ei3, z128)
    hp2, st2 = _tc_mm_stats(2, (agg1a, agg1b), (h1a, h1b), deg16, W_l1, W_r1, b1)
    h2a, h2b, y2 = _tc_norm2(hp2, st2, g2, be2, W_l2)

    p2a, p2b = _sc_agg_split()(y2, ei3, z128)
    z, u = _tc_mm_final((p2a, p2b), (h2a, h2b), deg16, W_r2, b2, W_bi[0])

    gs, gd = _sc_decoder_gather()(u, z, eli)
    return _tc_decoder_dot(gs, gd, b_bi)
